# baseline (speedup 1.0000x reference)
import functools

import jax
import jax.numpy as jnp
from jax import lax
from jax.experimental import pallas as pl
from jax.experimental.pallas import tpu as pltpu
from jax.experimental.pallas import tpu_sc as plsc

NUM_TABLES = 26
VOCAB_SZ = 100000
BATCH_SZ = 16384

NUM_CORES = 2
NUM_SUBCORES = 16
LANES = 16

CHUNK = 4096
NCHUNK = BATCH_SZ // CHUNK


def _emb_body(tables_hbm, idx_hbm, out_hbm,
              table_v, ibuf0, ibuf1, obuf0, obuf1, sem_t, sem_i, sem_o):
    wid = lax.axis_index("s") * NUM_CORES + lax.axis_index("c")

    @pl.when(wid < NUM_TABLES)
    def _():
        ibufs = (ibuf0, ibuf1)
        obufs = (obuf0, obuf1)
        tdesc = pltpu.async_copy(tables_hbm.at[wid], table_v, sem_t)
        descs_i = [
            pltpu.async_copy(
                idx_hbm.at[wid, pl.ds(q * CHUNK, CHUNK)], ibufs[q], sem_i)
            for q in range(2)
        ]
        tdesc.wait()
        descs_o = []
        for q in range(NCHUNK):
            ib, ob = ibufs[q % 2], obufs[q % 2]
            descs_i[q].wait()
            if q >= 2:
                descs_o[q - 2].wait()

            @pl.loop(0, CHUNK // LANES, unroll=16)
            def _gather(i):
                sl = pl.ds(i * LANES, LANES)
                ob[sl] = plsc.load_gather(table_v, [ib[sl]])

            descs_o.append(pltpu.async_copy(
                ob, out_hbm.at[wid, pl.ds(q * CHUNK, CHUNK)], sem_o))
            if q + 2 < NCHUNK:
                descs_i.append(pltpu.async_copy(
                    idx_hbm.at[wid, pl.ds((q + 2) * CHUNK, CHUNK)],
                    ib, sem_i))
        descs_o[NCHUNK - 2].wait()
        descs_o[NCHUNK - 1].wait()


@functools.partial(
    pl.kernel,
    out_type=jax.ShapeDtypeStruct((NUM_TABLES, BATCH_SZ), jnp.float32),
    mesh=plsc.VectorSubcoreMesh(core_axis_name="c", subcore_axis_name="s"),
    scratch_types=[
        pltpu.VMEM((VOCAB_SZ,), jnp.float32),
        pltpu.VMEM((CHUNK,), jnp.int32),
        pltpu.VMEM((CHUNK,), jnp.int32),
        pltpu.VMEM((CHUNK,), jnp.float32),
        pltpu.VMEM((CHUNK,), jnp.float32),
        pltpu.SemaphoreType.DMA,
        pltpu.SemaphoreType.DMA,
        pltpu.SemaphoreType.DMA,
    ],
    compiler_params=pltpu.CompilerParams(needs_layout_passes=False),
)
def _emb_kernel(tables_hbm, idx_hbm, out_hbm, *scratch):
    _emb_body(tables_hbm, idx_hbm, out_hbm, *scratch)


def kernel(categorical_tensor, tables):
    idx_t = categorical_tensor.astype(jnp.int32).T
    out_t = _emb_kernel(tables.reshape(NUM_TABLES, VOCAB_SZ), idx_t)
    return out_t.T
